# baseline (speedup 1.0000x reference)
import functools

import jax
import jax.numpy as jnp
from jax import lax
from jax.experimental import pallas as pl
from jax.experimental.pallas import tpu as pltpu
from jax.experimental.pallas import tpu_sc as plsc

N = 1024
G = 4
L = 16
NC, NS = 2, 16
NW = NC * NS
ROWS = N // NW
PLANE = N * N

_mesh = plsc.VectorSubcoreMesh(core_axis_name="c", subcore_axis_name="s")


@functools.partial(
    pl.kernel,
    mesh=_mesh,
    out_type=jax.ShapeDtypeStruct((2, G * PLANE), jnp.int32),
    scratch_types=[
        pltpu.VMEM((ROWS * N,), jnp.int32),
        pltpu.VMEM((ROWS * N,), jnp.int32),
        pltpu.SemaphoreType.DMA,
    ],
)
def _enumerate_pairs(out_hbm, rows_v, ramp_v, sem):
    wid = lax.axis_index("s") * NC + lax.axis_index("c")
    lane = lax.iota(jnp.int32, L)
    vecs_per_row = N // L

    def fill_rows(j, c):
        val = (ROWS * wid + j) + jnp.zeros((L,), jnp.int32)
        base = j * N
        for k in range(vecs_per_row):
            rows_v[pl.ds(base + k * L, L)] = val
        return c

    CH = 8
    copies = []
    for t in range(ROWS // CH):
        lax.fori_loop(t * CH, (t + 1) * CH, fill_rows, 0)
        src = rows_v.at[pl.ds(t * CH * N, CH * N)]
        for g in range(G):
            dst = out_hbm.at[
                0, pl.ds(g * PLANE + (ROWS * wid + t * CH) * N, CH * N)
            ]
            copies.append(pltpu.async_copy(src, dst, sem))

    def fill_ramp(j, c):
        vec = lane
        base = j * N
        for k in range(vecs_per_row):
            ramp_v[pl.ds(base + k * L, L)] = vec
            vec = vec + L
        return c

    lax.fori_loop(0, ROWS, fill_ramp, 0)

    col_base = (G * ROWS * wid) * N
    for c in range(G):
        dst = out_hbm.at[1, pl.ds(col_base + c * ROWS * N, ROWS * N)]
        copies.append(pltpu.async_copy(ramp_v, dst, sem))

    for cp in copies:
        cp.wait()


def kernel(x, edge_index, batch):
    return _enumerate_pairs()


# speedup vs baseline: 1.0013x; 1.0013x over previous
import functools

import jax
import jax.numpy as jnp
from jax import lax
from jax.experimental import pallas as pl
from jax.experimental.pallas import tpu as pltpu
from jax.experimental.pallas import tpu_sc as plsc

N = 1024
G = 4
L = 16
NC, NS = 2, 16
NW = NC * NS
ROWS = N // NW
PLANE = N * N

_mesh = plsc.VectorSubcoreMesh(core_axis_name="c", subcore_axis_name="s")


@functools.partial(
    pl.kernel,
    mesh=_mesh,
    out_type=jax.ShapeDtypeStruct((2, G * PLANE), jnp.int32),
    scratch_types=[
        pltpu.VMEM((ROWS * N,), jnp.int32),
        pltpu.VMEM((2 * ROWS * N,), jnp.int32),
        pltpu.SemaphoreType.DMA,
    ],
)
def _enumerate_pairs(out_hbm, rows_v, ramp_v, sem):
    wid = lax.axis_index("s") * NC + lax.axis_index("c")
    lane = lax.iota(jnp.int32, L)
    vecs_per_row = N // L

    def fill_rows(j, c):
        val = (ROWS * wid + j) + jnp.zeros((L,), jnp.int32)
        base = j * N
        for k in range(vecs_per_row):
            rows_v[pl.ds(base + k * L, L)] = val
        return c

    lax.fori_loop(0, ROWS, fill_rows, 0)

    copies = []
    for g in range(G):
        dst = out_hbm.at[0, pl.ds(g * PLANE + (ROWS * wid) * N, ROWS * N)]
        copies.append(pltpu.async_copy(rows_v, dst, sem))

    def fill_ramp(j, c):
        vec = lane
        base = j * N
        for k in range(vecs_per_row):
            ramp_v[pl.ds(base + k * L, L)] = vec
            vec = vec + L
        return c

    lax.fori_loop(0, 2 * ROWS, fill_ramp, 0)

    col_base = (G * ROWS * wid) * N
    for c in range(G // 2):
        dst = out_hbm.at[1, pl.ds(col_base + c * 2 * ROWS * N, 2 * ROWS * N)]
        copies.append(pltpu.async_copy(ramp_v, dst, sem))

    for cp in copies:
        cp.wait()


def kernel(x, edge_index, batch):
    return _enumerate_pairs()
